# baseline (speedup 1.0000x reference)
import functools

import jax
import jax.numpy as jnp
from jax.experimental import pallas as pl
from jax.experimental.pallas import tpu as pltpu

C = 384
H = 224
W = 224
S = 16
NH = H // S
NW = W // S
K = NH * NW
G = 16
KG = (NH + 2) * G
WIN = 3 * G
PB = S * W
NEG = -1e30
INV_SQRT_C = float(1.0 / (C ** 0.5))
F32 = jnp.float32
BF16 = jnp.bfloat16


NKEEP = 10


def _spin_kernel(x_ref, poolw_ref, ones_ref,
                 wq_ref, wk_ref, wv_ref, wo_ref, y_ref,
                 cent_scr, num_scr, den_scr, pix_scr,
                 ks_scr, vs_scr, prev_scr):
    i = pl.program_id(0)

    @pl.when(i == 0)
    def _():
        cent_scr[pl.ds(0, G), :] = jnp.zeros((G, C), F32)
        cent_scr[pl.ds(KG - G, G), :] = jnp.zeros((G, C), F32)
        num_scr[...] = jnp.zeros((KG, C), F32)
        den_scr[...] = jnp.zeros((KG, 8), F32)

    @pl.when(i < NH)
    def _():
        rowsum = jnp.sum(x_ref[0], axis=0)
        poolT = jax.lax.dot_general(
            poolw_ref[...], rowsum, (((0,), (0,)), ((), ())),
            preferred_element_type=F32)
        cent_scr[pl.ds((i + 1) * G, G), :] = poolT

    @pl.when((i >= 1) & (i <= NH))
    def _():
        bh = i - 1
        centw = cent_scr[pl.ds(bh * G, WIN), :]
        prev = prev_scr[...]
        dots = jax.lax.dot_general(
            prev, centw.astype(BF16), (((1,), (1,)), ((), ())),
            preferred_element_type=F32)
        csq = jnp.sum(centw * centw, axis=1)[None, :]
        r = jax.lax.broadcasted_iota(jnp.int32, (1, WIN), 1)
        kh = bh - 1 + r // G
        pen = jnp.where((kh >= 0) & (kh < NH), 0.0, -NEG)
        kwv = jax.lax.broadcasted_iota(jnp.int32, (1, WIN), 1) % G
        bwv = (jax.lax.broadcasted_iota(jnp.int32, (PB, 1), 0) % W) // S
        okc = (jnp.abs(kwv - bwv) <= 1) & (kwv < NW)
        lm = jnp.where(okc, 2.0 * dots - (csq + pen), NEG)
        m = jnp.max(lm, axis=1, keepdims=True)
        e = jnp.exp(lm - m)
        den = jnp.sum(e, axis=1, keepdims=True)
        aff = (e / den).astype(BF16)
        contrib = jax.lax.dot_general(
            aff, prev, (((0,), (0,)), ((), ())),
            preferred_element_type=F32)
        dcon = jax.lax.dot_general(
            aff, ones_ref[...], (((0,), (0,)), ((), ())),
            preferred_element_type=F32)
        num_scr[pl.ds(bh * G, WIN), :] += contrib
        den_scr[pl.ds(bh * G, WIN), :] += dcon

    @pl.when(i < NH)
    def _():
        pixm = x_ref[0].astype(BF16).reshape(PB, C)
        prev_scr[...] = pixm

        @pl.when(i < NKEEP)
        def _():
            pix_scr[i] = pixm

    @pl.when(i == NH)
    def _():
        cent1 = (num_scr[...] /
                 (den_scr[...][:, :1] + 1e-16)).astype(BF16)
        ks_scr[...] = jax.lax.dot_general(
            cent1, wk_ref[...].astype(BF16), (((1,), (0,)), ((), ())),
            preferred_element_type=F32).astype(BF16)
        vs_scr[...] = jax.lax.dot_general(
            cent1, wv_ref[...].astype(BF16), (((1,), (0,)), ((), ())),
            preferred_element_type=F32).astype(BF16)

    @pl.when(i > NH)
    def _():
        j = i - NH - 1
        pixj = jnp.where(
            j < NKEEP, pix_scr[jnp.minimum(j, NKEEP - 1)],
            x_ref[0].astype(BF16).reshape(PB, C))
        q = jax.lax.dot_general(
            pixj, wq_ref[...].astype(BF16), (((1,), (0,)), ((), ())),
            preferred_element_type=F32)
        logits = jax.lax.dot_general(
            q.astype(BF16), ks_scr[...], (((1,), (1,)), ((), ())),
            preferred_element_type=F32) * INV_SQRT_C
        r = jax.lax.broadcasted_iota(jnp.int32, (1, KG), 1)
        colmask = jnp.where((r >= G) & (r < KG - G) & (r % G < NW), 0.0, NEG)
        lm = logits + colmask
        m = jnp.max(lm, axis=1, keepdims=True)
        e = jnp.exp(lm - m)
        attn = (e / jnp.sum(e, axis=1, keepdims=True)).astype(BF16)
        out = jax.lax.dot_general(
            attn, vs_scr[...], (((1,), (0,)), ((), ())),
            preferred_element_type=F32)
        proj = jax.lax.dot_general(
            out.astype(BF16), wo_ref[...].astype(BF16), (((1,), (0,)), ((), ())),
            preferred_element_type=F32)
        y_ref[0] = (pixj.astype(F32) + proj).reshape(S, W, C)


@functools.partial(jax.jit, static_argnames=("interpret",))
def kernel(x, Wq, Wk, Wv, Wo, interpret=False):
    xt = x.transpose(0, 2, 3, 1)
    poolw = ((jnp.arange(W)[:, None] // S) ==
             jnp.arange(G)[None, :]).astype(F32) / (S * S)
    ones8 = jnp.ones((PB, 8), BF16)

    y = pl.pallas_call(
        _spin_kernel,
        grid=(2 * NH + 1,),
        in_specs=[
            pl.BlockSpec((1, S, W, C), lambda i: (
                0,
                jnp.where(i <= NH, jnp.minimum(i, NH - 1),
                          jnp.where(i - NH - 1 >= NKEEP, i - NH - 1, NH - 1)),
                0, 0)),
            pl.BlockSpec((W, G), lambda i: (0, 0)),
            pl.BlockSpec((PB, 8), lambda i: (0, 0)),
            pl.BlockSpec((C, C), lambda i: (0, 0)),
            pl.BlockSpec((C, C), lambda i: (0, 0)),
            pl.BlockSpec((C, C), lambda i: (0, 0)),
            pl.BlockSpec((C, C), lambda i: (0, 0)),
        ],
        out_specs=pl.BlockSpec(
            (1, S, W, C), lambda i: (0, jnp.maximum(i - NH - 1, 0), 0, 0)),
        out_shape=jax.ShapeDtypeStruct((1, H, W, C), F32),
        scratch_shapes=[
            pltpu.VMEM((KG, C), F32),
            pltpu.VMEM((KG, C), F32),
            pltpu.VMEM((KG, 8), F32),
            pltpu.VMEM((NKEEP, PB, C), BF16),
            pltpu.VMEM((KG, C), BF16),
            pltpu.VMEM((KG, C), BF16),
            pltpu.VMEM((PB, C), BF16),
        ],
        compiler_params=pltpu.CompilerParams(
            dimension_semantics=("arbitrary",),
            vmem_limit_bytes=100 * 1024 * 1024),
        interpret=interpret,
    )(xt, poolw, ones8, Wq, Wk, Wv, Wo)

    return y.transpose(0, 3, 1, 2)
